# baseline (speedup 1.0000x reference)
import jax
import jax.numpy as jnp
from jax.experimental import pallas as pl
from jax.experimental.pallas import tpu as pltpu

_MEM_ROWS = 100000
_MEM_UNIT = 256
_D_OUT = 768
_D_ALL = 1027
_CH = 10000
_NST = _MEM_ROWS // _CH
_NI = 4
_LAG = 2
_MIN_SIM = 0.5
_NEG_INF = float("-inf")
_IMAX = 0x7FFFFFFF


def _in_cp(mem, bufs, isem, step):
    b = step % _NI
    return pltpu.make_async_copy(
        mem.at[pl.ds(step * _CH, _CH)], bufs.at[b], isem.at[b])


def _out_cp(bufs, memo, osem, step):
    b = step % _NI
    return pltpu.make_async_copy(
        bufs.at[b], memo.at[pl.ds(step * _CH, _CH)], osem.at[b])


def _ntm_body(hp_ref, xj_ref, b_ref, w_hbm, mem_ref, y_ref, nr_ref, memo_ref,
              wbuf, ibufs, rowb, wsem, isem, osem, rsem,
              bs_ref, bi_ref):
    hp = hp_ref[0]

    w_cp = pltpu.make_async_copy(w_hbm, wbuf, wsem)
    w_cp.start()
    row_cp = pltpu.make_async_copy(mem_ref.at[pl.ds(hp, 1)], rowb, rsem)
    row_cp.start()
    for st in range(_NI):
        _in_cp(mem_ref, ibufs, isem, st).start()

    w_cp.wait()
    out_row = jax.lax.dot_general(
        xj_ref[...], wbuf[...], (((1,), (1,)), ((), ())),
        preferred_element_type=jnp.float32,
        precision=jax.lax.Precision.DEFAULT) + b_ref[...]
    y_ref[...] = out_row
    s = out_row[0, _D_OUT]
    j = out_row[0, _D_OUT + 1]
    w = out_row[0, _D_OUT + 2]
    m = out_row[:, _D_OUT + 3:]

    row_cp.wait()
    row_new = jnp.where(w > 0.5, m, rowb[...])
    rowb[...] = row_new
    dhp = row_new - m
    sim_hp = 1.0 - jnp.sqrt(jnp.sum(dhp * dhp)) / _MEM_UNIT

    bs_ref[0] = _NEG_INF
    bi_ref[0] = _IMAX

    for st in range(_NST):
        bi_n = st % _NI
        _in_cp(mem_ref, ibufs, isem, st).wait()
        blk = ibufs[bi_n]
        _out_cp(ibufs, memo_ref, osem, st).start()
        st_old = st - _LAG
        if st_old >= 0 and st_old + _NI < _NST:
            _out_cp(ibufs, memo_ref, osem, st_old).wait()
            _in_cp(mem_ref, ibufs, isem, st_old + _NI).start()

        rows = jax.lax.broadcasted_iota(jnp.int32, (_CH, 1), 0) + st * _CH
        d = blk - m
        d2 = jnp.sum(d * d, axis=1, keepdims=True)
        sims = 1.0 - jnp.sqrt(d2) / _MEM_UNIT
        sims = jnp.where(rows == hp, _NEG_INF, sims)
        bmax = jnp.max(sims)
        barg = jnp.min(jnp.where(sims == bmax, rows, _IMAX))

        @pl.when(bmax > bs_ref[0])
        def _upd():
            bs_ref[0] = bmax
            bi_ref[0] = barg

    for st in range(_NST - _NI, _NST):
        _out_cp(ibufs, memo_ref, osem, st).wait()

    wr_cp = pltpu.make_async_copy(rowb, memo_ref.at[pl.ds(hp, 1)], rsem)
    wr_cp.start()

    bs = bs_ref[0]
    bi = bi_ref[0]
    hp_wins = (sim_hp > bs) | ((sim_hp == bs) & (hp < bi))
    best_sim = jnp.where(hp_wins, sim_hp, bs)
    best_idx = jnp.where(hp_wins, hp, bi)
    jumped = jnp.where(best_sim > _MIN_SIM, best_idx, 0)
    head0 = jnp.where(j > 0.5, jumped, hp)
    shift = (s * 3.0 - 1e-9).astype(jnp.int32) - 1
    head = jnp.mod(head0 + shift, _MEM_ROWS)

    wr_cp.wait()
    rd_cp = pltpu.make_async_copy(memo_ref.at[pl.ds(head, 1)], rowb, rsem)
    rd_cp.start()
    rd_cp.wait()
    nr_ref[...] = rowb[...]


def kernel(x, prev_read, mem, W, b, head_pos):
    xj = jnp.concatenate([x, prev_read], axis=0)[None, :]
    hp = jnp.asarray(head_pos, jnp.int32).reshape(1)

    y2d, new_read, mem_out = pl.pallas_call(
        _ntm_body,
        in_specs=[
            pl.BlockSpec(memory_space=pltpu.MemorySpace.SMEM),
            pl.BlockSpec((1, 1024), lambda: (0, 0)),
            pl.BlockSpec((1, _D_ALL), lambda: (0, 0)),
            pl.BlockSpec(memory_space=pltpu.MemorySpace.HBM),
            pl.BlockSpec(memory_space=pltpu.MemorySpace.HBM),
        ],
        out_specs=[
            pl.BlockSpec((1, _D_ALL), lambda: (0, 0)),
            pl.BlockSpec((1, _MEM_UNIT), lambda: (0, 0)),
            pl.BlockSpec(memory_space=pltpu.MemorySpace.HBM),
        ],
        out_shape=[
            jax.ShapeDtypeStruct((1, _D_ALL), jnp.float32),
            jax.ShapeDtypeStruct((1, _MEM_UNIT), jnp.float32),
            jax.ShapeDtypeStruct((_MEM_ROWS, _MEM_UNIT), jnp.float32),
        ],
        scratch_shapes=[
            pltpu.VMEM((_D_ALL, 1024), jnp.float32),
            pltpu.VMEM((_NI, _CH, _MEM_UNIT), jnp.float32),
            pltpu.VMEM((1, _MEM_UNIT), jnp.float32),
            pltpu.SemaphoreType.DMA,
            pltpu.SemaphoreType.DMA((_NI,)),
            pltpu.SemaphoreType.DMA((_NI,)),
            pltpu.SemaphoreType.DMA,
            pltpu.SMEM((1,), jnp.float32),
            pltpu.SMEM((1,), jnp.int32),
        ],
    )(hp, xj, b[None, :], W, mem)

    return (y2d[0, :_D_OUT], new_read.reshape(_MEM_UNIT), mem_out)


# speedup vs baseline: 1.0052x; 1.0052x over previous
import jax
import jax.numpy as jnp
from jax.experimental import pallas as pl
from jax.experimental.pallas import tpu as pltpu

_MEM_ROWS = 100000
_MEM_UNIT = 256
_D_OUT = 768
_D_ALL = 1027
_CH = 5000
_NST = _MEM_ROWS // _CH
_NI = 8
_LAG = 2
_MIN_SIM = 0.5
_NEG_INF = float("-inf")
_IMAX = 0x7FFFFFFF


def _in_cp(mem, bufs, isem, step):
    b = step % _NI
    return pltpu.make_async_copy(
        mem.at[pl.ds(step * _CH, _CH)], bufs.at[b], isem.at[b])


def _out_cp(bufs, memo, osem, step):
    b = step % _NI
    return pltpu.make_async_copy(
        bufs.at[b], memo.at[pl.ds(step * _CH, _CH)], osem.at[b])


def _ntm_body(hp_ref, xj_ref, b_ref, w_hbm, mem_ref, y_ref, nr_ref, memo_ref,
              wbuf, ibufs, rowb, wsem, isem, osem, rsem,
              bs_ref, bi_ref):
    hp = hp_ref[0]

    w_cp = pltpu.make_async_copy(w_hbm, wbuf, wsem)
    w_cp.start()
    row_cp = pltpu.make_async_copy(mem_ref.at[pl.ds(hp, 1)], rowb, rsem)
    row_cp.start()
    for st in range(_NI):
        _in_cp(mem_ref, ibufs, isem, st).start()

    w_cp.wait()
    out_row = jax.lax.dot_general(
        xj_ref[...], wbuf[...], (((1,), (1,)), ((), ())),
        preferred_element_type=jnp.float32,
        precision=jax.lax.Precision.DEFAULT) + b_ref[...]
    y_ref[...] = out_row
    s = out_row[0, _D_OUT]
    j = out_row[0, _D_OUT + 1]
    w = out_row[0, _D_OUT + 2]
    m = out_row[:, _D_OUT + 3:]

    row_cp.wait()
    row_new = jnp.where(w > 0.5, m, rowb[...])
    rowb[...] = row_new
    dhp = row_new - m
    sim_hp = 1.0 - jnp.sqrt(jnp.sum(dhp * dhp)) / _MEM_UNIT

    bs_ref[0] = _NEG_INF
    bi_ref[0] = _IMAX

    for st in range(_NST):
        bi_n = st % _NI
        _in_cp(mem_ref, ibufs, isem, st).wait()
        blk = ibufs[bi_n]
        _out_cp(ibufs, memo_ref, osem, st).start()
        st_old = st - _LAG
        if st_old >= 0 and st_old + _NI < _NST:
            _out_cp(ibufs, memo_ref, osem, st_old).wait()
            _in_cp(mem_ref, ibufs, isem, st_old + _NI).start()

        rows = jax.lax.broadcasted_iota(jnp.int32, (_CH, 1), 0) + st * _CH
        d = blk - m
        d2 = jnp.sum(d * d, axis=1, keepdims=True)
        sims = 1.0 - jnp.sqrt(d2) / _MEM_UNIT
        sims = jnp.where(rows == hp, _NEG_INF, sims)
        bmax = jnp.max(sims)
        barg = jnp.min(jnp.where(sims == bmax, rows, _IMAX))

        @pl.when(bmax > bs_ref[0])
        def _upd():
            bs_ref[0] = bmax
            bi_ref[0] = barg

    for st in range(_NST - _NI, _NST):
        _out_cp(ibufs, memo_ref, osem, st).wait()

    wr_cp = pltpu.make_async_copy(rowb, memo_ref.at[pl.ds(hp, 1)], rsem)
    wr_cp.start()

    bs = bs_ref[0]
    bi = bi_ref[0]
    hp_wins = (sim_hp > bs) | ((sim_hp == bs) & (hp < bi))
    best_sim = jnp.where(hp_wins, sim_hp, bs)
    best_idx = jnp.where(hp_wins, hp, bi)
    jumped = jnp.where(best_sim > _MIN_SIM, best_idx, 0)
    head0 = jnp.where(j > 0.5, jumped, hp)
    shift = (s * 3.0 - 1e-9).astype(jnp.int32) - 1
    head = jnp.mod(head0 + shift, _MEM_ROWS)

    wr_cp.wait()
    rd_cp = pltpu.make_async_copy(memo_ref.at[pl.ds(head, 1)], rowb, rsem)
    rd_cp.start()
    rd_cp.wait()
    nr_ref[...] = rowb[...]


def kernel(x, prev_read, mem, W, b, head_pos):
    xj = jnp.concatenate([x, prev_read], axis=0)[None, :]
    hp = jnp.asarray(head_pos, jnp.int32).reshape(1)

    y2d, new_read, mem_out = pl.pallas_call(
        _ntm_body,
        in_specs=[
            pl.BlockSpec(memory_space=pltpu.MemorySpace.SMEM),
            pl.BlockSpec((1, 1024), lambda: (0, 0)),
            pl.BlockSpec((1, _D_ALL), lambda: (0, 0)),
            pl.BlockSpec(memory_space=pltpu.MemorySpace.HBM),
            pl.BlockSpec(memory_space=pltpu.MemorySpace.HBM),
        ],
        out_specs=[
            pl.BlockSpec((1, _D_ALL), lambda: (0, 0)),
            pl.BlockSpec((1, _MEM_UNIT), lambda: (0, 0)),
            pl.BlockSpec(memory_space=pltpu.MemorySpace.HBM),
        ],
        out_shape=[
            jax.ShapeDtypeStruct((1, _D_ALL), jnp.float32),
            jax.ShapeDtypeStruct((1, _MEM_UNIT), jnp.float32),
            jax.ShapeDtypeStruct((_MEM_ROWS, _MEM_UNIT), jnp.float32),
        ],
        scratch_shapes=[
            pltpu.VMEM((_D_ALL, 1024), jnp.float32),
            pltpu.VMEM((_NI, _CH, _MEM_UNIT), jnp.float32),
            pltpu.VMEM((1, _MEM_UNIT), jnp.float32),
            pltpu.SemaphoreType.DMA,
            pltpu.SemaphoreType.DMA((_NI,)),
            pltpu.SemaphoreType.DMA((_NI,)),
            pltpu.SemaphoreType.DMA,
            pltpu.SMEM((1,), jnp.float32),
            pltpu.SMEM((1,), jnp.int32),
        ],
    )(hp, xj, b[None, :], W, mem)

    return (y2d[0, :_D_OUT], new_read.reshape(_MEM_UNIT), mem_out)
